# Initial kernel scaffold; baseline (speedup 1.0000x reference)
import jax
import jax.numpy as jnp
from jax.experimental import pallas as pl


def kernel(x, batch, W1, b1, g1, be1, W2, b2, g2, be2, W3, b3, g3, be3, W4, b4, g4, be4, Wl, bl):
    raise NotImplementedError("write your pallas kernel here")


# speedup vs baseline: 3.6647x; 3.6647x over previous
import functools

import jax
import jax.numpy as jnp
from jax import lax
from jax.experimental import pallas as pl
from jax.experimental.pallas import tpu as pltpu
from jax.experimental.pallas import tpu_sc as plsc

N = 10000
NPAD = 10240
KNN = 20
NGRAPH = 10
EPAD = NPAD * KNN
E_REAL = N * KNN

RB = 256
CB = 128
EB = 2048
NB = 512


def _knn_body(c0_ref, nc_ref, xr_ref, xt_ref, br_ref, bc_ref, idx_ref, dist_ref):
    rb = pl.program_id(0)
    c0 = c0_ref[rb]
    nc = nc_ref[rb]
    xr = xr_ref[...]
    brow = br_ref[...]
    xxr = jnp.sum(xr * xr, axis=1, keepdims=True)

    def dist_body(j, _):
        cols = pl.multiple_of(c0 + j * CB, CB)
        xc = xt_ref[:, pl.ds(cols, CB)]
        bc = bc_ref[:, pl.ds(cols, CB)]
        d = (xxr - 2.0 * jnp.dot(xr, xc, preferred_element_type=jnp.float32)
             + jnp.sum(xc * xc, axis=0, keepdims=True))
        d = jnp.where(brow == bc, d, jnp.inf)
        dist_ref[:, pl.ds(pl.multiple_of(j * CB, CB), CB)] = d
        return 0

    lax.fori_loop(0, nc, dist_body, 0, unroll=False)

    lane = lax.broadcasted_iota(jnp.int32, (1, CB), 1)
    inf = jnp.float32(jnp.inf)
    cols_out = []
    prev = jnp.full((RB, 1), -1, jnp.int32)
    for k in range(KNN):
        def scan_body(j, carry, prev=prev, mask_prev=(k > 0)):
            best, bidx = carry
            loc = pl.multiple_of(j * CB, CB)
            gi = lane + (c0 + j * CB)
            d = dist_ref[:, pl.ds(loc, CB)]
            if mask_prev:
                d = jnp.where(gi == prev, inf, d)
                dist_ref[:, pl.ds(loc, CB)] = d
            m = jnp.min(d, axis=1, keepdims=True)
            ai = jnp.min(jnp.where(d == m, gi, NPAD), axis=1, keepdims=True)
            better = (m < best) | ((m == best) & (ai < bidx))
            return (jnp.where(better, m, best), jnp.where(better, ai, bidx))

        best0 = jnp.full((RB, 1), inf, jnp.float32)
        bidx0 = jnp.full((RB, 1), NPAD, jnp.int32)
        _, bidx = lax.fori_loop(0, nc, scan_body, (best0, bidx0), unroll=False)
        cols_out.append(bidx)
        prev = bidx

    idx_ref[...] = jnp.concatenate(cols_out, axis=1)


def _knn(xpad, xt, bcol, brow, c0s, ncs, f):
    nblk = NPAD // RB
    grid_spec = pltpu.PrefetchScalarGridSpec(
        num_scalar_prefetch=2,
        grid=(nblk,),
        in_specs=[
            pl.BlockSpec((RB, f), lambda i, s0, s1: (i, 0)),
            pl.BlockSpec((f, NPAD), lambda i, s0, s1: (0, 0)),
            pl.BlockSpec((RB, 1), lambda i, s0, s1: (i, 0)),
            pl.BlockSpec((1, NPAD), lambda i, s0, s1: (0, 0)),
        ],
        out_specs=pl.BlockSpec((RB, KNN), lambda i, s0, s1: (i, 0)),
        scratch_shapes=[pltpu.VMEM((RB, NPAD), jnp.float32)],
    )
    return pl.pallas_call(
        _knn_body,
        grid_spec=grid_spec,
        out_shape=jax.ShapeDtypeStruct((NPAD, KNN), jnp.int32),
    )(c0s, ncs, xpad, xt, brow, bcol)


def _sc_gather(table, idx, d, chunk):
    info = plsc.get_sparse_core_info()
    ncores, nsub = info.num_cores, info.num_subcores
    nw = ncores * nsub
    b_per_w = EPAD // nw
    nchunk = b_per_w // chunk

    mesh = plsc.VectorSubcoreMesh(core_axis_name="c", subcore_axis_name="s")

    @functools.partial(
        pl.kernel,
        out_type=jax.ShapeDtypeStruct((EPAD, d), jnp.float32),
        mesh=mesh,
        compiler_params=pltpu.CompilerParams(use_tc_tiling_on_sc=False),
        scratch_types=[
            pltpu.VMEM((chunk,), jnp.int32),
            pltpu.VMEM((chunk, d), jnp.float32),
            pltpu.SemaphoreType.DMA,
        ],
    )
    def gather_kernel(table_hbm, idx_hbm, out_hbm, idx_v, rows_v, sem):
        wid = lax.axis_index("s") * ncores + lax.axis_index("c")
        base = wid * b_per_w
        for c in range(nchunk):
            off = base + c * chunk
            pltpu.sync_copy(idx_hbm.at[pl.ds(off, chunk)], idx_v)
            pltpu.async_copy(table_hbm.at[idx_v], rows_v, sem).wait()
            pltpu.sync_copy(rows_v, out_hbm.at[pl.ds(off, chunk)])

    return gather_kernel(table, idx)


def _mlp2_body(xi_ref, xj_ref, w_ref, b_ref, h_ref, s_ref):
    i = pl.program_id(0)
    xi = xi_ref[...]
    xj = xj_ref[..., :xi.shape[1]]
    e = jnp.concatenate([xi, xj - xi], axis=1)
    h = jnp.dot(e, w_ref[...], preferred_element_type=jnp.float32) + b_ref[...]
    h = jnp.maximum(h, 0.0)
    h_ref[...] = h
    e0 = i * EB + lax.broadcasted_iota(jnp.int32, (EB, 1), 0)
    hm = jnp.where(e0 < E_REAL, h, 0.0)
    fo = h.shape[1]
    s_ref[...] = jnp.sum(hm, axis=0).reshape(1, 1, fo)


def _mlp1_body(x_ref, m_ref, r_ref, g_ref, be_ref, w_ref, b_ref, h_ref, s_ref):
    i = pl.program_id(0)
    hb = g_ref[...] * (x_ref[...] - m_ref[...]) * r_ref[...] + be_ref[...]
    h = jnp.dot(hb, w_ref[...], preferred_element_type=jnp.float32) + b_ref[...]
    h = jnp.maximum(h, 0.0)
    h_ref[...] = h
    e0 = i * EB + lax.broadcasted_iota(jnp.int32, (EB, 1), 0)
    hm = jnp.where(e0 < E_REAL, h, 0.0)
    fo = h.shape[1]
    s_ref[...] = jnp.sum(hm, axis=0).reshape(1, 1, fo)


def _var_body(h_ref, m_ref, v_ref):
    i = pl.program_id(0)
    c = h_ref[...] - m_ref[...]
    e0 = i * EB + lax.broadcasted_iota(jnp.int32, (EB, 1), 0)
    c = jnp.where(e0 < E_REAL, c, 0.0)
    fo = c.shape[1]
    v_ref[...] = jnp.sum(c * c, axis=0).reshape(1, 1, fo)


def _mlp_pass2(xi, xj, w, b):
    fi, fo = w.shape
    fj = xj.shape[1]
    nblk = EPAD // EB
    h, s = pl.pallas_call(
        _mlp2_body,
        grid=(nblk,),
        in_specs=[
            pl.BlockSpec((EB, fi // 2), lambda i: (i, 0)),
            pl.BlockSpec((EB, fj), lambda i: (i, 0)),
            pl.BlockSpec((fi, fo), lambda i: (0, 0)),
            pl.BlockSpec((1, fo), lambda i: (0, 0)),
        ],
        out_specs=[
            pl.BlockSpec((EB, fo), lambda i: (i, 0)),
            pl.BlockSpec((1, 1, fo), lambda i: (i, 0, 0)),
        ],
        out_shape=[
            jax.ShapeDtypeStruct((EPAD, fo), jnp.float32),
            jax.ShapeDtypeStruct((nblk, 1, fo), jnp.float32),
        ],
    )(xi, xj, w, b.reshape(1, fo))
    return h, s


def _mlp_pass1(x, m, r, g, be, w, b):
    fi, fo = w.shape
    nblk = EPAD // EB
    vec = pl.BlockSpec((1, fi), lambda i: (0, 0))
    h, s = pl.pallas_call(
        _mlp1_body,
        grid=(nblk,),
        in_specs=[
            pl.BlockSpec((EB, fi), lambda i: (i, 0)),
            vec, vec, vec, vec,
            pl.BlockSpec((fi, fo), lambda i: (0, 0)),
            pl.BlockSpec((1, fo), lambda i: (0, 0)),
        ],
        out_specs=[
            pl.BlockSpec((EB, fo), lambda i: (i, 0)),
            pl.BlockSpec((1, 1, fo), lambda i: (i, 0, 0)),
        ],
        out_shape=[
            jax.ShapeDtypeStruct((EPAD, fo), jnp.float32),
            jax.ShapeDtypeStruct((nblk, 1, fo), jnp.float32),
        ],
    )(x, m.reshape(1, fi), r.reshape(1, fi), g.reshape(1, fi),
      be.reshape(1, fi), w, b.reshape(1, fo))
    return h, s


def _bn_stats(h, s_blk):
    fo = h.shape[1]
    nblk = EPAD // EB
    mean = jnp.sum(s_blk, axis=0)[0] / E_REAL
    v_blk = pl.pallas_call(
        _var_body,
        grid=(nblk,),
        in_specs=[
            pl.BlockSpec((EB, fo), lambda i: (i, 0)),
            pl.BlockSpec((1, fo), lambda i: (0, 0)),
        ],
        out_specs=pl.BlockSpec((1, 1, fo), lambda i: (i, 0, 0)),
        out_shape=jax.ShapeDtypeStruct((nblk, 1, fo), jnp.float32),
    )(h, mean.reshape(1, fo))
    var = jnp.sum(v_blk, axis=0)[0] / E_REAL
    return mean, lax.rsqrt(var + 1e-5)


def _max_body(h_ref, m_ref, r_ref, g_ref, be_ref, o_ref):
    hmax = h_ref[:, 0, :]
    hmin = hmax
    for j in range(1, KNN):
        hj = h_ref[:, j, :]
        hmax = jnp.maximum(hmax, hj)
        hmin = jnp.minimum(hmin, hj)
    g = g_ref[...]
    sel = jnp.where(g * r_ref[...] >= 0.0, hmax, hmin)
    o_ref[...] = g * (sel - m_ref[...]) * r_ref[...] + be_ref[...]


def _maxk(h3d, m, r, g, be, fo):
    nblk = NPAD // NB
    vec = pl.BlockSpec((1, fo), lambda i: (0, 0))
    return pl.pallas_call(
        _max_body,
        grid=(nblk,),
        in_specs=[
            pl.BlockSpec((NB, KNN, fo), lambda i: (i, 0, 0)),
            vec, vec, vec, vec,
        ],
        out_specs=pl.BlockSpec((NB, fo), lambda i: (i, 0)),
        out_shape=jax.ShapeDtypeStruct((NPAD, fo), jnp.float32),
    )(h3d, m.reshape(1, fo), r.reshape(1, fo), g.reshape(1, fo),
      be.reshape(1, fo))


def _pool_body(h_ref, b_ref, wl_ref, bl_ref, pooled_ref, out_ref):
    i = pl.program_id(0)
    ninf = jnp.float32(-jnp.inf)

    @pl.when(i == 0)
    def _():
        pooled_ref[...] = jnp.full_like(pooled_ref, ninf)

    h = h_ref[...]
    b = b_ref[...]
    for g in range(NGRAPH):
        m = jnp.max(jnp.where(b == g, h, ninf), axis=0, keepdims=True)
        pooled_ref[pl.ds(g, 1), :] = jnp.maximum(pooled_ref[pl.ds(g, 1), :], m)

    @pl.when(i == pl.num_programs(0) - 1)
    def _():
        out_ref[...] = (jnp.dot(pooled_ref[...], wl_ref[...],
                                preferred_element_type=jnp.float32) + bl_ref[...])


def _pool(h2, brow, wl, bl):
    fo = h2.shape[1]
    nblk = NPAD // NB
    _, out = pl.pallas_call(
        _pool_body,
        grid=(nblk,),
        in_specs=[
            pl.BlockSpec((NB, fo), lambda i: (i, 0)),
            pl.BlockSpec((NB, 1), lambda i: (i, 0)),
            pl.BlockSpec((fo, 2), lambda i: (0, 0)),
            pl.BlockSpec((1, 2), lambda i: (0, 0)),
        ],
        out_specs=[
            pl.BlockSpec((NGRAPH, fo), lambda i: (0, 0)),
            pl.BlockSpec((NGRAPH, 2), lambda i: (0, 0)),
        ],
        out_shape=[
            jax.ShapeDtypeStruct((NGRAPH, fo), jnp.float32),
            jax.ShapeDtypeStruct((NGRAPH, 2), jnp.float32),
        ],
    )(h2, brow, wl, bl.reshape(1, 2))
    return out


def _row_block_ranges(batch):
    gids = jnp.arange(NGRAPH, dtype=batch.dtype)
    starts = jnp.searchsorted(batch, gids, side='left').astype(jnp.int32)
    ends = jnp.searchsorted(batch, gids, side='right').astype(jnp.int32)
    sizes = ends - starts

    r0 = jnp.minimum(jnp.arange(NPAD // RB, dtype=jnp.int32) * RB, N - 1)
    r1 = jnp.minimum(r0 + RB - 1, N - 1)
    bmin = batch[r0].astype(jnp.int32)
    bmax = batch[r1].astype(jnp.int32)

    span = (gids[None, :].astype(jnp.int32) >= bmin[:, None]) & (
        gids[None, :].astype(jnp.int32) <= bmax[:, None])
    minsize = jnp.min(jnp.where(span, sizes[None, :], N), axis=1)
    degenerate = minsize < KNN

    c0 = (starts[bmin] // CB) * CB
    c1 = ends[bmax]
    c0 = jnp.where(degenerate, 0, c0).astype(jnp.int32)
    c1 = jnp.where(degenerate, N, c1).astype(jnp.int32)
    nc = (c1 - c0 + CB - 1) // CB
    return c0, nc.astype(jnp.int32)


def kernel(x, batch, W1, b1, g1, be1, W2, b2, g2, be2, W3, b3, g3, be3,
           W4, b4, g4, be4, Wl, bl):
    batch = batch.astype(jnp.int32)
    xpad = jnp.concatenate(
        [x, jnp.zeros((NPAD - N, 8), jnp.float32)], axis=0)
    bpad = jnp.concatenate(
        [batch, jnp.full((NPAD - N,), -1, jnp.int32)], axis=0)
    bcol = bpad.reshape(1, NPAD)
    brow = bpad.reshape(NPAD, 1)
    c0s, ncs = _row_block_ranges(batch)

    idx1 = _knn(xpad, xpad.T, bcol, brow, c0s, ncs, 8)
    x16 = jnp.concatenate([xpad, jnp.zeros((NPAD, 8), jnp.float32)], axis=1)
    xj1 = _sc_gather(x16, idx1.reshape(EPAD), 16, EPAD // 32)
    xi1 = jnp.broadcast_to(xpad[:, None, :], (NPAD, KNN, 8)).reshape(EPAD, 8)

    h1a, s = _mlp_pass2(xi1, xj1, W1, b1)
    m1, r1 = _bn_stats(h1a, s)

    h1b, s = _mlp_pass1(h1a, m1, r1, g1, be1, W2, b2)
    m2, r2 = _bn_stats(h1b, s)

    h1c, s = _mlp_pass1(h1b, m2, r2, g2, be2, W3, b3)
    m3, r3 = _bn_stats(h1c, s)

    h1 = _maxk(h1c.reshape(NPAD, KNN, 64), m3, r3, g3, be3, 64)

    idx2 = _knn(h1, h1.T, bcol, brow, c0s, ncs, 64)
    xj2 = _sc_gather(h1, idx2.reshape(EPAD), 64, EPAD // 32 // 4)
    xi2 = jnp.broadcast_to(h1[:, None, :], (NPAD, KNN, 64)).reshape(EPAD, 64)

    h2a, s = _mlp_pass2(xi2, xj2, W4, b4)
    m4, r4 = _bn_stats(h2a, s)

    h2 = _maxk(h2a.reshape(NPAD, KNN, 128), m4, r4, g4, be4, 128)

    return _pool(h2, brow, Wl, bl)


# speedup vs baseline: 7.8816x; 2.1507x over previous
import functools

import jax
import jax.numpy as jnp
from jax import lax
from jax.experimental import pallas as pl
from jax.experimental.pallas import tpu as pltpu
from jax.experimental.pallas import tpu_sc as plsc

N = 10000
NPAD = 10240
KNN = 20
NGRAPH = 10
EPAD = NPAD * KNN
E_REAL = N * KNN

RB = 512
CB = 512
EB = 2048
NB = 512


def _knn_body(c0_ref, nc_ref, xr_ref, xt_ref, br_ref, bc_ref, idx_ref, dist_ref):
    rb = pl.program_id(0)
    c0 = c0_ref[rb]
    nc = nc_ref[rb]
    xr = xr_ref[...]
    brow = br_ref[...]
    xxr = jnp.sum(xr * xr, axis=1, keepdims=True)

    def dist_body(j, _):
        cols = pl.multiple_of(c0 + j * CB, CB)
        xc = xt_ref[:, pl.ds(cols, CB)]
        bc = bc_ref[:, pl.ds(cols, CB)]
        d = (xxr - 2.0 * jnp.dot(xr, xc, preferred_element_type=jnp.float32)
             + jnp.sum(xc * xc, axis=0, keepdims=True))
        d = jnp.where(brow == bc, d, jnp.inf)
        dist_ref[:, pl.ds(pl.multiple_of(j * CB, CB), CB)] = d
        return 0

    lax.fori_loop(0, nc, dist_body, 0, unroll=False)

    lane = lax.broadcasted_iota(jnp.int32, (1, CB), 1)
    inf = jnp.float32(jnp.inf)
    cols_out = []
    prev = jnp.full((RB, 1), -1, jnp.int32)
    for k in range(KNN):
        def scan_body(j, carry, prev=prev, mask_prev=(k > 0)):
            best, bidx = carry
            loc = pl.multiple_of(j * CB, CB)
            gi = lane + (c0 + j * CB)
            d = dist_ref[:, pl.ds(loc, CB)]
            if mask_prev:
                d = jnp.where(gi == prev, inf, d)
                dist_ref[:, pl.ds(loc, CB)] = d
            m = jnp.min(d, axis=1, keepdims=True)
            ai = jnp.min(jnp.where(d == m, gi, NPAD), axis=1, keepdims=True)
            better = (m < best) | ((m == best) & (ai < bidx))
            return (jnp.where(better, m, best), jnp.where(better, ai, bidx))

        best0 = jnp.full((RB, 1), inf, jnp.float32)
        bidx0 = jnp.full((RB, 1), NPAD, jnp.int32)
        _, bidx = lax.fori_loop(0, nc, scan_body, (best0, bidx0), unroll=False)
        cols_out.append(bidx)
        prev = bidx

    idx_ref[...] = jnp.concatenate(cols_out, axis=1)


def _knn(xpad, xt, bcol, brow, c0s, ncs, f):
    nblk = NPAD // RB
    grid_spec = pltpu.PrefetchScalarGridSpec(
        num_scalar_prefetch=2,
        grid=(nblk,),
        in_specs=[
            pl.BlockSpec((RB, f), lambda i, s0, s1: (i, 0)),
            pl.BlockSpec((f, NPAD), lambda i, s0, s1: (0, 0)),
            pl.BlockSpec((RB, 1), lambda i, s0, s1: (i, 0)),
            pl.BlockSpec((1, NPAD), lambda i, s0, s1: (0, 0)),
        ],
        out_specs=pl.BlockSpec((RB, KNN), lambda i, s0, s1: (i, 0)),
        scratch_shapes=[pltpu.VMEM((RB, NPAD), jnp.float32)],
    )
    return pl.pallas_call(
        _knn_body,
        grid_spec=grid_spec,
        out_shape=jax.ShapeDtypeStruct((NPAD, KNN), jnp.int32),
    )(c0s, ncs, xpad, xt, brow, bcol)


def _sc_gather(table, idx, d, chunk):
    info = plsc.get_sparse_core_info()
    ncores, nsub = info.num_cores, info.num_subcores
    nw = ncores * nsub
    b_per_w = EPAD // nw
    nchunk = b_per_w // chunk

    mesh = plsc.VectorSubcoreMesh(core_axis_name="c", subcore_axis_name="s")

    @functools.partial(
        pl.kernel,
        out_type=jax.ShapeDtypeStruct((EPAD, d), jnp.float32),
        mesh=mesh,
        compiler_params=pltpu.CompilerParams(use_tc_tiling_on_sc=False),
        scratch_types=[
            pltpu.VMEM((chunk,), jnp.int32),
            pltpu.VMEM((chunk, d), jnp.float32),
            pltpu.SemaphoreType.DMA,
        ],
    )
    def gather_kernel(table_hbm, idx_hbm, out_hbm, idx_v, rows_v, sem):
        wid = lax.axis_index("s") * ncores + lax.axis_index("c")
        base = wid * b_per_w
        for c in range(nchunk):
            off = base + c * chunk
            pltpu.sync_copy(idx_hbm.at[pl.ds(off, chunk)], idx_v)
            pltpu.async_copy(table_hbm.at[idx_v], rows_v, sem).wait()
            pltpu.sync_copy(rows_v, out_hbm.at[pl.ds(off, chunk)])

    return gather_kernel(table, idx)


def _mlp2_body(xi_ref, xj_ref, w_ref, b_ref, h_ref, s_ref):
    i = pl.program_id(0)
    xi = xi_ref[...]
    xj = xj_ref[..., :xi.shape[1]]
    e = jnp.concatenate([xi, xj - xi], axis=1)
    h = jnp.dot(e, w_ref[...], preferred_element_type=jnp.float32) + b_ref[...]
    h = jnp.maximum(h, 0.0)
    h_ref[...] = h
    e0 = i * EB + lax.broadcasted_iota(jnp.int32, (EB, 1), 0)
    hm = jnp.where(e0 < E_REAL, h, 0.0)
    fo = h.shape[1]
    s_ref[...] = jnp.sum(hm, axis=0).reshape(1, 1, fo)


def _mlp1_body(x_ref, m_ref, r_ref, g_ref, be_ref, w_ref, b_ref, h_ref, s_ref):
    i = pl.program_id(0)
    hb = g_ref[...] * (x_ref[...] - m_ref[...]) * r_ref[...] + be_ref[...]
    h = jnp.dot(hb, w_ref[...], preferred_element_type=jnp.float32) + b_ref[...]
    h = jnp.maximum(h, 0.0)
    h_ref[...] = h
    e0 = i * EB + lax.broadcasted_iota(jnp.int32, (EB, 1), 0)
    hm = jnp.where(e0 < E_REAL, h, 0.0)
    fo = h.shape[1]
    s_ref[...] = jnp.sum(hm, axis=0).reshape(1, 1, fo)


def _var_body(h_ref, m_ref, v_ref):
    i = pl.program_id(0)
    c = h_ref[...] - m_ref[...]
    e0 = i * EB + lax.broadcasted_iota(jnp.int32, (EB, 1), 0)
    c = jnp.where(e0 < E_REAL, c, 0.0)
    fo = c.shape[1]
    v_ref[...] = jnp.sum(c * c, axis=0).reshape(1, 1, fo)


def _mlp_pass2(xi, xj, w, b):
    fi, fo = w.shape
    fj = xj.shape[1]
    nblk = EPAD // EB
    h, s = pl.pallas_call(
        _mlp2_body,
        grid=(nblk,),
        in_specs=[
            pl.BlockSpec((EB, fi // 2), lambda i: (i, 0)),
            pl.BlockSpec((EB, fj), lambda i: (i, 0)),
            pl.BlockSpec((fi, fo), lambda i: (0, 0)),
            pl.BlockSpec((1, fo), lambda i: (0, 0)),
        ],
        out_specs=[
            pl.BlockSpec((EB, fo), lambda i: (i, 0)),
            pl.BlockSpec((1, 1, fo), lambda i: (i, 0, 0)),
        ],
        out_shape=[
            jax.ShapeDtypeStruct((EPAD, fo), jnp.float32),
            jax.ShapeDtypeStruct((nblk, 1, fo), jnp.float32),
        ],
    )(xi, xj, w, b.reshape(1, fo))
    return h, s


def _mlp_pass1(x, m, r, g, be, w, b):
    fi, fo = w.shape
    nblk = EPAD // EB
    vec = pl.BlockSpec((1, fi), lambda i: (0, 0))
    h, s = pl.pallas_call(
        _mlp1_body,
        grid=(nblk,),
        in_specs=[
            pl.BlockSpec((EB, fi), lambda i: (i, 0)),
            vec, vec, vec, vec,
            pl.BlockSpec((fi, fo), lambda i: (0, 0)),
            pl.BlockSpec((1, fo), lambda i: (0, 0)),
        ],
        out_specs=[
            pl.BlockSpec((EB, fo), lambda i: (i, 0)),
            pl.BlockSpec((1, 1, fo), lambda i: (i, 0, 0)),
        ],
        out_shape=[
            jax.ShapeDtypeStruct((EPAD, fo), jnp.float32),
            jax.ShapeDtypeStruct((nblk, 1, fo), jnp.float32),
        ],
    )(x, m.reshape(1, fi), r.reshape(1, fi), g.reshape(1, fi),
      be.reshape(1, fi), w, b.reshape(1, fo))
    return h, s


def _bn_stats(h, s_blk):
    fo = h.shape[1]
    nblk = EPAD // EB
    mean = jnp.sum(s_blk, axis=0)[0] / E_REAL
    v_blk = pl.pallas_call(
        _var_body,
        grid=(nblk,),
        in_specs=[
            pl.BlockSpec((EB, fo), lambda i: (i, 0)),
            pl.BlockSpec((1, fo), lambda i: (0, 0)),
        ],
        out_specs=pl.BlockSpec((1, 1, fo), lambda i: (i, 0, 0)),
        out_shape=jax.ShapeDtypeStruct((nblk, 1, fo), jnp.float32),
    )(h, mean.reshape(1, fo))
    var = jnp.sum(v_blk, axis=0)[0] / E_REAL
    return mean, lax.rsqrt(var + 1e-5)


def _max_body(h_ref, m_ref, r_ref, g_ref, be_ref, o_ref):
    hmax = h_ref[:, 0, :]
    hmin = hmax
    for j in range(1, KNN):
        hj = h_ref[:, j, :]
        hmax = jnp.maximum(hmax, hj)
        hmin = jnp.minimum(hmin, hj)
    g = g_ref[...]
    sel = jnp.where(g * r_ref[...] >= 0.0, hmax, hmin)
    o_ref[...] = g * (sel - m_ref[...]) * r_ref[...] + be_ref[...]


def _maxk(h3d, m, r, g, be, fo):
    nblk = NPAD // NB
    vec = pl.BlockSpec((1, fo), lambda i: (0, 0))
    return pl.pallas_call(
        _max_body,
        grid=(nblk,),
        in_specs=[
            pl.BlockSpec((NB, KNN, fo), lambda i: (i, 0, 0)),
            vec, vec, vec, vec,
        ],
        out_specs=pl.BlockSpec((NB, fo), lambda i: (i, 0)),
        out_shape=jax.ShapeDtypeStruct((NPAD, fo), jnp.float32),
    )(h3d, m.reshape(1, fo), r.reshape(1, fo), g.reshape(1, fo),
      be.reshape(1, fo))


def _pool_body(h_ref, b_ref, wl_ref, bl_ref, pooled_ref, out_ref):
    i = pl.program_id(0)
    ninf = jnp.float32(-jnp.inf)

    @pl.when(i == 0)
    def _():
        pooled_ref[...] = jnp.full_like(pooled_ref, ninf)

    h = h_ref[...]
    b = b_ref[...]
    for g in range(NGRAPH):
        m = jnp.max(jnp.where(b == g, h, ninf), axis=0, keepdims=True)
        pooled_ref[pl.ds(g, 1), :] = jnp.maximum(pooled_ref[pl.ds(g, 1), :], m)

    @pl.when(i == pl.num_programs(0) - 1)
    def _():
        out_ref[...] = (jnp.dot(pooled_ref[...], wl_ref[...],
                                preferred_element_type=jnp.float32) + bl_ref[...])


def _pool(h2, brow, wl, bl):
    fo = h2.shape[1]
    nblk = NPAD // NB
    _, out = pl.pallas_call(
        _pool_body,
        grid=(nblk,),
        in_specs=[
            pl.BlockSpec((NB, fo), lambda i: (i, 0)),
            pl.BlockSpec((NB, 1), lambda i: (i, 0)),
            pl.BlockSpec((fo, 2), lambda i: (0, 0)),
            pl.BlockSpec((1, 2), lambda i: (0, 0)),
        ],
        out_specs=[
            pl.BlockSpec((NGRAPH, fo), lambda i: (0, 0)),
            pl.BlockSpec((NGRAPH, 2), lambda i: (0, 0)),
        ],
        out_shape=[
            jax.ShapeDtypeStruct((NGRAPH, fo), jnp.float32),
            jax.ShapeDtypeStruct((NGRAPH, 2), jnp.float32),
        ],
    )(h2, brow, wl, bl.reshape(1, 2))
    return out


def _row_block_ranges(batch):
    gids = jnp.arange(NGRAPH, dtype=batch.dtype)
    starts = jnp.searchsorted(batch, gids, side='left').astype(jnp.int32)
    ends = jnp.searchsorted(batch, gids, side='right').astype(jnp.int32)
    sizes = ends - starts

    r0 = jnp.minimum(jnp.arange(NPAD // RB, dtype=jnp.int32) * RB, N - 1)
    r1 = jnp.minimum(r0 + RB - 1, N - 1)
    bmin = batch[r0].astype(jnp.int32)
    bmax = batch[r1].astype(jnp.int32)

    span = (gids[None, :].astype(jnp.int32) >= bmin[:, None]) & (
        gids[None, :].astype(jnp.int32) <= bmax[:, None])
    minsize = jnp.min(jnp.where(span, sizes[None, :], N), axis=1)
    degenerate = minsize < KNN

    c0 = (starts[bmin] // CB) * CB
    c1 = ends[bmax]
    c0 = jnp.where(degenerate, 0, c0).astype(jnp.int32)
    c1 = jnp.where(degenerate, N, c1).astype(jnp.int32)
    nc = (c1 - c0 + CB - 1) // CB
    return c0, nc.astype(jnp.int32)


def kernel(x, batch, W1, b1, g1, be1, W2, b2, g2, be2, W3, b3, g3, be3,
           W4, b4, g4, be4, Wl, bl):
    batch = batch.astype(jnp.int32)
    xpad = jnp.concatenate(
        [x, jnp.zeros((NPAD - N, 8), jnp.float32)], axis=0)
    bpad = jnp.concatenate(
        [batch, jnp.full((NPAD - N,), -1, jnp.int32)], axis=0)
    bcol = bpad.reshape(1, NPAD)
    brow = bpad.reshape(NPAD, 1)
    c0s, ncs = _row_block_ranges(batch)

    idx1 = _knn(xpad, xpad.T, bcol, brow, c0s, ncs, 8)
    x16 = jnp.concatenate([xpad, jnp.zeros((NPAD, 8), jnp.float32)], axis=1)
    xj1 = _sc_gather(x16, idx1.reshape(EPAD), 16, EPAD // 32)
    xi1 = jnp.broadcast_to(xpad[:, None, :], (NPAD, KNN, 8)).reshape(EPAD, 8)

    h1a, s = _mlp_pass2(xi1, xj1, W1, b1)
    m1, r1 = _bn_stats(h1a, s)

    h1b, s = _mlp_pass1(h1a, m1, r1, g1, be1, W2, b2)
    m2, r2 = _bn_stats(h1b, s)

    h1c, s = _mlp_pass1(h1b, m2, r2, g2, be2, W3, b3)
    m3, r3 = _bn_stats(h1c, s)

    h1 = _maxk(h1c.reshape(NPAD, KNN, 64), m3, r3, g3, be3, 64)

    idx2 = _knn(h1, h1.T, bcol, brow, c0s, ncs, 64)
    xj2 = _sc_gather(h1, idx2.reshape(EPAD), 64, EPAD // 32 // 4)
    xi2 = jnp.broadcast_to(h1[:, None, :], (NPAD, KNN, 64)).reshape(EPAD, 64)

    h2a, s = _mlp_pass2(xi2, xj2, W4, b4)
    m4, r4 = _bn_stats(h2a, s)

    h2 = _maxk(h2a.reshape(NPAD, KNN, 128), m4, r4, g4, be4, 128)

    return _pool(h2, brow, Wl, bl)


# speedup vs baseline: 8.0775x; 1.0249x over previous
import functools

import jax
import jax.numpy as jnp
from jax import lax
from jax.experimental import pallas as pl
from jax.experimental.pallas import tpu as pltpu
from jax.experimental.pallas import tpu_sc as plsc

N = 10000
NPAD = 10240
KNN = 20
NGRAPH = 10
EPAD = NPAD * KNN
E_REAL = N * KNN

RB = 512
CB = 512
EB = 2048
NB = 512


def _knn_body(c0_ref, nc_ref, xr_ref, xt_ref, br_ref, bc_ref, idx_ref, dist_ref):
    rb = pl.program_id(0)
    c0 = c0_ref[rb]
    nc = nc_ref[rb]
    xr = xr_ref[...]
    brow = br_ref[...]
    xxr = jnp.sum(xr * xr, axis=1, keepdims=True)

    def dist_body(j, _):
        cols = pl.multiple_of(c0 + j * CB, CB)
        xc = xt_ref[:, pl.ds(cols, CB)]
        bc = bc_ref[:, pl.ds(cols, CB)]
        d = (xxr - 2.0 * jnp.dot(xr, xc, preferred_element_type=jnp.float32)
             + jnp.sum(xc * xc, axis=0, keepdims=True))
        d = jnp.where(brow == bc, d, jnp.inf)
        dist_ref[:, pl.ds(pl.multiple_of(j * CB, CB), CB)] = d
        return 0

    lax.fori_loop(0, nc, dist_body, 0, unroll=False)

    lane = lax.broadcasted_iota(jnp.int32, (1, CB), 1)
    inf = jnp.float32(jnp.inf)
    cols_out = []
    prev = jnp.full((RB, 1), -1, jnp.int32)
    for k in range(KNN):
        def scan_body(j, carry, prev=prev, mask_prev=(k > 0)):
            best, bidx = carry
            loc = pl.multiple_of(j * CB, CB)
            gi = lane + (c0 + j * CB)
            d = dist_ref[:, pl.ds(loc, CB)]
            if mask_prev:
                d = jnp.where(gi == prev, inf, d)
                dist_ref[:, pl.ds(loc, CB)] = d
            m = jnp.min(d, axis=1, keepdims=True)
            ai = jnp.min(jnp.where(d == m, gi, NPAD), axis=1, keepdims=True)
            better = (m < best) | ((m == best) & (ai < bidx))
            return (jnp.where(better, m, best), jnp.where(better, ai, bidx))

        best0 = jnp.full((RB, 1), inf, jnp.float32)
        bidx0 = jnp.full((RB, 1), NPAD, jnp.int32)
        _, bidx = lax.fori_loop(0, nc, scan_body, (best0, bidx0), unroll=False)
        cols_out.append(bidx)
        prev = bidx

    idx_ref[...] = jnp.concatenate(cols_out, axis=1)


def _knn(xpad, xt, bcol, brow, c0s, ncs, f):
    nblk = NPAD // RB
    grid_spec = pltpu.PrefetchScalarGridSpec(
        num_scalar_prefetch=2,
        grid=(nblk,),
        in_specs=[
            pl.BlockSpec((RB, f), lambda i, s0, s1: (i, 0)),
            pl.BlockSpec((f, NPAD), lambda i, s0, s1: (0, 0)),
            pl.BlockSpec((RB, 1), lambda i, s0, s1: (i, 0)),
            pl.BlockSpec((1, NPAD), lambda i, s0, s1: (0, 0)),
        ],
        out_specs=pl.BlockSpec((RB, KNN), lambda i, s0, s1: (i, 0)),
        scratch_shapes=[pltpu.VMEM((RB, NPAD), jnp.float32)],
    )
    return pl.pallas_call(
        _knn_body,
        grid_spec=grid_spec,
        out_shape=jax.ShapeDtypeStruct((NPAD, KNN), jnp.int32),
    )(c0s, ncs, xpad, xt, brow, bcol)


def _sc_gather(table, idx, d, chunk):
    info = plsc.get_sparse_core_info()
    ncores, nsub = info.num_cores, info.num_subcores
    nw = ncores * nsub
    b_per_w = EPAD // nw
    nchunk = b_per_w // chunk

    mesh = plsc.VectorSubcoreMesh(core_axis_name="c", subcore_axis_name="s")

    @functools.partial(
        pl.kernel,
        out_type=jax.ShapeDtypeStruct((EPAD, d), jnp.float32),
        mesh=mesh,
        compiler_params=pltpu.CompilerParams(use_tc_tiling_on_sc=False),
        scratch_types=[
            pltpu.VMEM((chunk,), jnp.int32),
            pltpu.VMEM((chunk, d), jnp.float32),
            pltpu.SemaphoreType.DMA,
        ],
    )
    def gather_kernel(table_hbm, idx_hbm, out_hbm, idx_v, rows_v, sem):
        wid = lax.axis_index("s") * ncores + lax.axis_index("c")
        base = wid * b_per_w
        for c in range(nchunk):
            off = base + c * chunk
            pltpu.sync_copy(idx_hbm.at[pl.ds(off, chunk)], idx_v)
            pltpu.async_copy(table_hbm.at[idx_v], rows_v, sem).wait()
            pltpu.sync_copy(rows_v, out_hbm.at[pl.ds(off, chunk)])

    return gather_kernel(table, idx)


def _mlp2_body(xi_ref, xj_ref, w_ref, b_ref, h_ref, s_ref):
    i = pl.program_id(0)
    xi = xi_ref[...]
    xj = xj_ref[..., :xi.shape[1]]
    e = jnp.concatenate([xi, xj - xi], axis=1)
    h = jnp.dot(e, w_ref[...], preferred_element_type=jnp.float32) + b_ref[...]
    h = jnp.maximum(h, 0.0)
    h_ref[...] = h
    e0 = i * EB + lax.broadcasted_iota(jnp.int32, (EB, 1), 0)
    hm = jnp.where(e0 < E_REAL, h, 0.0)
    fo = h.shape[1]
    s_ref[...] = jnp.sum(hm, axis=0).reshape(1, 1, fo)


def _mlp1_body(x_ref, m_ref, r_ref, g_ref, be_ref, w_ref, b_ref, h_ref, s_ref):
    i = pl.program_id(0)
    hb = g_ref[...] * (x_ref[...] - m_ref[...]) * r_ref[...] + be_ref[...]
    h = jnp.dot(hb, w_ref[...], preferred_element_type=jnp.float32) + b_ref[...]
    h = jnp.maximum(h, 0.0)
    h_ref[...] = h
    e0 = i * EB + lax.broadcasted_iota(jnp.int32, (EB, 1), 0)
    hm = jnp.where(e0 < E_REAL, h, 0.0)
    fo = h.shape[1]
    s_ref[...] = jnp.sum(hm, axis=0).reshape(1, 1, fo)


NB2 = 128
EB2 = NB2 * KNN


def _mlp2n_body(xin_ref, xj_ref, w_ref, b_ref, h_ref, s_ref):
    i = pl.program_id(0)
    xi_n = xin_ref[...]
    f = xi_n.shape[1]
    xi = jnp.broadcast_to(xi_n[:, None, :], (NB2, KNN, f)).reshape(EB2, f)
    e = jnp.concatenate([xi, xj_ref[...] - xi], axis=1)
    h = jnp.dot(e, w_ref[...], preferred_element_type=jnp.float32) + b_ref[...]
    h = jnp.maximum(h, 0.0)
    h_ref[...] = h
    e0 = i * EB2 + lax.broadcasted_iota(jnp.int32, (EB2, 1), 0)
    hm = jnp.where(e0 < E_REAL, h, 0.0)
    fo = h.shape[1]
    s_ref[...] = jnp.sum(hm, axis=0).reshape(1, 1, fo)


def _mlp_pass2n(xnode, xj, w, b):
    fi, fo = w.shape
    f = fi // 2
    nblk = EPAD // EB2
    h, s = pl.pallas_call(
        _mlp2n_body,
        grid=(nblk,),
        in_specs=[
            pl.BlockSpec((NB2, f), lambda i: (i, 0)),
            pl.BlockSpec((EB2, f), lambda i: (i, 0)),
            pl.BlockSpec((fi, fo), lambda i: (0, 0)),
            pl.BlockSpec((1, fo), lambda i: (0, 0)),
        ],
        out_specs=[
            pl.BlockSpec((EB2, fo), lambda i: (i, 0)),
            pl.BlockSpec((1, 1, fo), lambda i: (i, 0, 0)),
        ],
        out_shape=[
            jax.ShapeDtypeStruct((EPAD, fo), jnp.float32),
            jax.ShapeDtypeStruct((nblk, 1, fo), jnp.float32),
        ],
    )(xnode, xj, w, b.reshape(1, fo))
    return h, s


def _var_body(h_ref, m_ref, v_ref):
    i = pl.program_id(0)
    c = h_ref[...] - m_ref[...]
    e0 = i * EB + lax.broadcasted_iota(jnp.int32, (EB, 1), 0)
    c = jnp.where(e0 < E_REAL, c, 0.0)
    fo = c.shape[1]
    v_ref[...] = jnp.sum(c * c, axis=0).reshape(1, 1, fo)


def _mlp_pass2(xi, xj, w, b):
    fi, fo = w.shape
    fj = xj.shape[1]
    nblk = EPAD // EB
    h, s = pl.pallas_call(
        _mlp2_body,
        grid=(nblk,),
        in_specs=[
            pl.BlockSpec((EB, fi // 2), lambda i: (i, 0)),
            pl.BlockSpec((EB, fj), lambda i: (i, 0)),
            pl.BlockSpec((fi, fo), lambda i: (0, 0)),
            pl.BlockSpec((1, fo), lambda i: (0, 0)),
        ],
        out_specs=[
            pl.BlockSpec((EB, fo), lambda i: (i, 0)),
            pl.BlockSpec((1, 1, fo), lambda i: (i, 0, 0)),
        ],
        out_shape=[
            jax.ShapeDtypeStruct((EPAD, fo), jnp.float32),
            jax.ShapeDtypeStruct((nblk, 1, fo), jnp.float32),
        ],
    )(xi, xj, w, b.reshape(1, fo))
    return h, s


def _mlp_pass1(x, m, r, g, be, w, b):
    fi, fo = w.shape
    nblk = EPAD // EB
    vec = pl.BlockSpec((1, fi), lambda i: (0, 0))
    h, s = pl.pallas_call(
        _mlp1_body,
        grid=(nblk,),
        in_specs=[
            pl.BlockSpec((EB, fi), lambda i: (i, 0)),
            vec, vec, vec, vec,
            pl.BlockSpec((fi, fo), lambda i: (0, 0)),
            pl.BlockSpec((1, fo), lambda i: (0, 0)),
        ],
        out_specs=[
            pl.BlockSpec((EB, fo), lambda i: (i, 0)),
            pl.BlockSpec((1, 1, fo), lambda i: (i, 0, 0)),
        ],
        out_shape=[
            jax.ShapeDtypeStruct((EPAD, fo), jnp.float32),
            jax.ShapeDtypeStruct((nblk, 1, fo), jnp.float32),
        ],
    )(x, m.reshape(1, fi), r.reshape(1, fi), g.reshape(1, fi),
      be.reshape(1, fi), w, b.reshape(1, fo))
    return h, s


def _bn_stats(h, s_blk):
    fo = h.shape[1]
    nblk = EPAD // EB
    mean = jnp.sum(s_blk, axis=0)[0] / E_REAL
    v_blk = pl.pallas_call(
        _var_body,
        grid=(nblk,),
        in_specs=[
            pl.BlockSpec((EB, fo), lambda i: (i, 0)),
            pl.BlockSpec((1, fo), lambda i: (0, 0)),
        ],
        out_specs=pl.BlockSpec((1, 1, fo), lambda i: (i, 0, 0)),
        out_shape=jax.ShapeDtypeStruct((nblk, 1, fo), jnp.float32),
    )(h, mean.reshape(1, fo))
    var = jnp.sum(v_blk, axis=0)[0] / E_REAL
    return mean, lax.rsqrt(var + 1e-5)


def _max_body(h_ref, m_ref, hx_ref, hn_ref, v_ref):
    i = pl.program_id(0)
    m = m_ref[...]
    n0 = i * NB + lax.broadcasted_iota(jnp.int32, (NB, 1), 0)
    real = n0 < N
    h0 = h_ref[:, 0, :]
    hmax = h0
    hmin = h0
    c = h0 - m
    acc = jnp.where(real, c * c, 0.0)
    for j in range(1, KNN):
        hj = h_ref[:, j, :]
        hmax = jnp.maximum(hmax, hj)
        hmin = jnp.minimum(hmin, hj)
        c = hj - m
        acc = acc + jnp.where(real, c * c, 0.0)
    hx_ref[...] = hmax
    hn_ref[...] = hmin
    fo = hmax.shape[1]
    v_ref[...] = jnp.sum(acc, axis=0).reshape(1, 1, fo)


def _maxk(h3d, mean, fo):
    nblk = NPAD // NB
    hx, hn, v = pl.pallas_call(
        _max_body,
        grid=(nblk,),
        in_specs=[
            pl.BlockSpec((NB, KNN, fo), lambda i: (i, 0, 0)),
            pl.BlockSpec((1, fo), lambda i: (0, 0)),
        ],
        out_specs=[
            pl.BlockSpec((NB, fo), lambda i: (i, 0)),
            pl.BlockSpec((NB, fo), lambda i: (i, 0)),
            pl.BlockSpec((1, 1, fo), lambda i: (i, 0, 0)),
        ],
        out_shape=[
            jax.ShapeDtypeStruct((NPAD, fo), jnp.float32),
            jax.ShapeDtypeStruct((NPAD, fo), jnp.float32),
            jax.ShapeDtypeStruct((nblk, 1, fo), jnp.float32),
        ],
    )(h3d, mean.reshape(1, fo))
    var = jnp.sum(v, axis=0)[0] / E_REAL
    return hx, hn, lax.rsqrt(var + 1e-5)


def _bn_sel_body(hx_ref, hn_ref, m_ref, r_ref, g_ref, be_ref, o_ref):
    g = g_ref[...]
    r = r_ref[...]
    sel = jnp.where(g * r >= 0.0, hx_ref[...], hn_ref[...])
    o_ref[...] = g * (sel - m_ref[...]) * r + be_ref[...]


def _bn_sel(hx, hn, m, r, g, be, fo):
    nblk = NPAD // NB
    vec = pl.BlockSpec((1, fo), lambda i: (0, 0))
    blk = pl.BlockSpec((NB, fo), lambda i: (i, 0))
    return pl.pallas_call(
        _bn_sel_body,
        grid=(nblk,),
        in_specs=[blk, blk, vec, vec, vec, vec],
        out_specs=blk,
        out_shape=jax.ShapeDtypeStruct((NPAD, fo), jnp.float32),
    )(hx, hn, m.reshape(1, fo), r.reshape(1, fo), g.reshape(1, fo),
      be.reshape(1, fo))


def _pool_body(hx_ref, hn_ref, b_ref, m_ref, r_ref, g_ref, be_ref,
               wl_ref, bl_ref, pmax_ref, pmin_ref, out_ref):
    i = pl.program_id(0)
    ninf = jnp.float32(-jnp.inf)
    pinf = jnp.float32(jnp.inf)

    @pl.when(i == 0)
    def _():
        pmax_ref[...] = jnp.full_like(pmax_ref, ninf)
        pmin_ref[...] = jnp.full_like(pmin_ref, pinf)

    hx = hx_ref[...]
    hn = hn_ref[...]
    b = b_ref[...]
    for g in range(NGRAPH):
        mx = jnp.max(jnp.where(b == g, hx, ninf), axis=0, keepdims=True)
        mn = jnp.min(jnp.where(b == g, hn, pinf), axis=0, keepdims=True)
        pmax_ref[pl.ds(g, 1), :] = jnp.maximum(pmax_ref[pl.ds(g, 1), :], mx)
        pmin_ref[pl.ds(g, 1), :] = jnp.minimum(pmin_ref[pl.ds(g, 1), :], mn)

    @pl.when(i == pl.num_programs(0) - 1)
    def _():
        gg = g_ref[...]
        rr = r_ref[...]
        sel = jnp.where(gg * rr >= 0.0, pmax_ref[...], pmin_ref[...])
        bn = gg * (sel - m_ref[...]) * rr + be_ref[...]
        out_ref[...] = (jnp.dot(bn, wl_ref[...],
                                preferred_element_type=jnp.float32) + bl_ref[...])


def _pool(hx, hn, brow, m, r, g, be, wl, bl):
    fo = hx.shape[1]
    nblk = NPAD // NB
    vec = pl.BlockSpec((1, fo), lambda i: (0, 0))
    blk = pl.BlockSpec((NB, fo), lambda i: (i, 0))
    _, _, out = pl.pallas_call(
        _pool_body,
        grid=(nblk,),
        in_specs=[
            blk, blk,
            pl.BlockSpec((NB, 1), lambda i: (i, 0)),
            vec, vec, vec, vec,
            pl.BlockSpec((fo, 2), lambda i: (0, 0)),
            pl.BlockSpec((1, 2), lambda i: (0, 0)),
        ],
        out_specs=[
            pl.BlockSpec((NGRAPH, fo), lambda i: (0, 0)),
            pl.BlockSpec((NGRAPH, fo), lambda i: (0, 0)),
            pl.BlockSpec((NGRAPH, 2), lambda i: (0, 0)),
        ],
        out_shape=[
            jax.ShapeDtypeStruct((NGRAPH, fo), jnp.float32),
            jax.ShapeDtypeStruct((NGRAPH, fo), jnp.float32),
            jax.ShapeDtypeStruct((NGRAPH, 2), jnp.float32),
        ],
    )(hx, hn, brow, m.reshape(1, fo), r.reshape(1, fo), g.reshape(1, fo),
      be.reshape(1, fo), wl, bl.reshape(1, 2))
    return out


def _row_block_ranges(batch):
    gids = jnp.arange(NGRAPH, dtype=batch.dtype)
    starts = jnp.searchsorted(batch, gids, side='left').astype(jnp.int32)
    ends = jnp.searchsorted(batch, gids, side='right').astype(jnp.int32)
    sizes = ends - starts

    r0 = jnp.minimum(jnp.arange(NPAD // RB, dtype=jnp.int32) * RB, N - 1)
    r1 = jnp.minimum(r0 + RB - 1, N - 1)
    bmin = batch[r0].astype(jnp.int32)
    bmax = batch[r1].astype(jnp.int32)

    span = (gids[None, :].astype(jnp.int32) >= bmin[:, None]) & (
        gids[None, :].astype(jnp.int32) <= bmax[:, None])
    minsize = jnp.min(jnp.where(span, sizes[None, :], N), axis=1)
    degenerate = minsize < KNN

    c0 = (starts[bmin] // CB) * CB
    c1 = ends[bmax]
    c0 = jnp.where(degenerate, 0, c0).astype(jnp.int32)
    c1 = jnp.where(degenerate, N, c1).astype(jnp.int32)
    nc = (c1 - c0 + CB - 1) // CB
    return c0, nc.astype(jnp.int32)


def kernel(x, batch, W1, b1, g1, be1, W2, b2, g2, be2, W3, b3, g3, be3,
           W4, b4, g4, be4, Wl, bl):
    batch = batch.astype(jnp.int32)
    xpad = jnp.concatenate(
        [x, jnp.zeros((NPAD - N, 8), jnp.float32)], axis=0)
    bpad = jnp.concatenate(
        [batch, jnp.full((NPAD - N,), -1, jnp.int32)], axis=0)
    bcol = bpad.reshape(1, NPAD)
    brow = bpad.reshape(NPAD, 1)
    c0s, ncs = _row_block_ranges(batch)

    idx1 = _knn(xpad, xpad.T, bcol, brow, c0s, ncs, 8)
    x16 = jnp.concatenate([xpad, jnp.zeros((NPAD, 8), jnp.float32)], axis=1)
    xj1 = _sc_gather(x16, idx1.reshape(EPAD), 16, EPAD // 32)
    xi1 = jnp.broadcast_to(xpad[:, None, :], (NPAD, KNN, 8)).reshape(EPAD, 8)

    h1a, s = _mlp_pass2(xi1, xj1, W1, b1)
    m1, r1 = _bn_stats(h1a, s)

    h1b, s = _mlp_pass1(h1a, m1, r1, g1, be1, W2, b2)
    m2, r2 = _bn_stats(h1b, s)

    h1c, s = _mlp_pass1(h1b, m2, r2, g2, be2, W3, b3)
    m3 = jnp.sum(s, axis=0)[0] / E_REAL
    hx1, hn1, r3 = _maxk(h1c.reshape(NPAD, KNN, 64), m3, 64)
    h1 = _bn_sel(hx1, hn1, m3, r3, g3, be3, 64)

    idx2 = _knn(h1, h1.T, bcol, brow, c0s, ncs, 64)
    xj2 = _sc_gather(h1, idx2.reshape(EPAD), 64, EPAD // 32 // 4)

    h2a, s = _mlp_pass2n(h1, xj2, W4, b4)
    m4 = jnp.sum(s, axis=0)[0] / E_REAL
    hx2, hn2, r4 = _maxk(h2a.reshape(NPAD, KNN, 128), m4, 128)

    return _pool(hx2, hn2, brow, m4, r4, g4, be4, Wl, bl)
